# baseline (device time: 91721 ns/iter reference)
import jax
import jax.numpy as jnp
from jax import lax
from jax.experimental import pallas as pl
from jax.experimental.pallas import tpu as pltpu

N_DEV = 4


def kernel(x, Win0, Wout0, Win1, Wout1, Win2, Wout2):
    m_per, d = x.shape
    m = N_DEV * m_per

    def body(x_ref, win0_ref, wout0_ref, win1_ref, wout1_ref, win2_ref,
             wout2_ref, out_ref, xfull, agc, arc,
             ag_send, ag_recv, ar_send, ar_recv):
        my = lax.axis_index("i")
        left = lax.rem(my - 1 + N_DEV, N_DEV)
        right = lax.rem(my + 1, N_DEV)

        barrier_sem = pltpu.get_barrier_semaphore()
        for nbr in (left, right):
            pl.semaphore_signal(
                barrier_sem, inc=1,
                device_id=(nbr,), device_id_type=pl.DeviceIdType.MESH,
            )
        pl.semaphore_wait(barrier_sem, 2)

        xfull[pl.ds(my * m_per, m_per), :] = x_ref[:, :]
        agc[0, :, :] = x_ref[:, :]
        for h in range(N_DEV - 1):
            rdma = pltpu.make_async_remote_copy(
                src_ref=agc.at[h],
                dst_ref=agc.at[h + 1],
                send_sem=ag_send.at[h],
                recv_sem=ag_recv.at[h],
                device_id=(right,),
                device_id_type=pl.DeviceIdType.MESH,
            )
            rdma.start()
            rdma.wait()
            origin = lax.rem(my - h - 1 + N_DEV, N_DEV)
            xfull[pl.ds(origin * m_per, m_per), :] = agc[h + 1, :, :]

        layers = ((win0_ref, wout0_ref), (win1_ref, wout1_ref),
                  (win2_ref, wout2_ref))
        for l, (win_ref, wout_ref) in enumerate(layers):
            hidden = jnp.maximum(
                jnp.dot(xfull[:, :], win_ref[:, :],
                        preferred_element_type=jnp.float32),
                0.0,
            )
            partial = jnp.dot(hidden, wout_ref[:, :],
                              preferred_element_type=jnp.float32)
            target = out_ref if l == N_LAYERS - 1 else xfull
            target[:, :] = partial
            arc[0, :, :] = partial
            for h in range(N_DEV - 1):
                rdma = pltpu.make_async_remote_copy(
                    src_ref=arc.at[h],
                    dst_ref=arc.at[h + 1],
                    send_sem=ar_send.at[l * (N_DEV - 1) + h],
                    recv_sem=ar_recv.at[l * (N_DEV - 1) + h],
                    device_id=(right,),
                    device_id_type=pl.DeviceIdType.MESH,
                )
                rdma.start()
                rdma.wait()
                target[:, :] = target[:, :] + arc[h + 1, :, :]

    N_LAYERS = 3
    return pl.pallas_call(
        body,
        out_shape=jax.ShapeDtypeStruct((m, d), jnp.float32),
        in_specs=[pl.BlockSpec(memory_space=pltpu.VMEM)] * 7,
        out_specs=pl.BlockSpec(memory_space=pltpu.VMEM),
        scratch_shapes=[
            pltpu.VMEM((m, d), jnp.float32),
            pltpu.VMEM((N_DEV, m_per, d), jnp.float32),
            pltpu.VMEM((N_DEV, m, d), jnp.float32),
            pltpu.SemaphoreType.DMA((N_DEV - 1,)),
            pltpu.SemaphoreType.DMA((N_DEV - 1,)),
            pltpu.SemaphoreType.DMA((N_LAYERS * (N_DEV - 1),)),
            pltpu.SemaphoreType.DMA((N_LAYERS * (N_DEV - 1),)),
        ],
        compiler_params=pltpu.CompilerParams(collective_id=0),
    )(x, Win0, Wout0, Win1, Wout1, Win2, Wout2)


# device time: 50905 ns/iter; 1.8018x vs baseline; 1.8018x over previous
import jax
import jax.numpy as jnp
from jax import lax
from jax.experimental import pallas as pl
from jax.experimental.pallas import tpu as pltpu

N_DEV = 4
N_LAYERS = 3
N_PEER = N_DEV - 1


def kernel(x, Win0, Wout0, Win1, Wout1, Win2, Wout2):
    m_per, d = x.shape
    m = N_DEV * m_per

    def body(x_ref, win0, wout0, win1, wout1, win2, wout2, out_ref,
             xbuf, rs, rs_src, obuf,
             ag_send, ag_recv, rs_send, rs_recv, out_send, out_recv):
        my = lax.axis_index("i")
        pending_sends = []

        def chunk_copy(src_ref, dst_ref, send_sem, recv_sem, peer):
            return pltpu.make_async_remote_copy(
                src_ref=src_ref, dst_ref=dst_ref,
                send_sem=send_sem, recv_sem=recv_sem,
                device_id=(peer,), device_id_type=pl.DeviceIdType.MESH,
            )

        def wait_chunk_recv(dst_ref, recv_sem):
            chunk_copy(x_ref, dst_ref, ag_send.at[0], recv_sem, my).wait_recv()

        barrier_sem = pltpu.get_barrier_semaphore()
        for k in range(1, N_DEV):
            pl.semaphore_signal(
                barrier_sem, inc=1,
                device_id=(lax.rem(my + k, N_DEV),),
                device_id_type=pl.DeviceIdType.MESH,
            )
        pl.semaphore_wait(barrier_sem, N_DEV - 1)

        xbuf[0, N_PEER] = x_ref[...]
        for k in range(1, N_DEV):
            peer = lax.rem(my + k, N_DEV)
            s = N_PEER - k
            rdma = chunk_copy(x_ref, xbuf.at[0, s],
                              ag_send.at[s], ag_recv.at[s], peer)
            rdma.start()
            pending_sends.append(rdma)

        weights = ((win0, wout0), (win1, wout1), (win2, wout2))

        def layer_partial(win, wout, xc):
            h = jnp.maximum(
                jnp.dot(xc, win[...], preferred_element_type=jnp.float32), 0.0)
            return jnp.dot(h, wout[...], preferred_element_type=jnp.float32)

        for l in range(N_LAYERS):
            win, wout = weights[l]
            par = l % 2
            for s in range(N_PEER):
                wait_chunk_recv(xbuf.at[par, s], ag_recv.at[l * 3 + s])
                partial = layer_partial(win, wout, xbuf[par, s])
                rs_src[l * 3 + s] = partial
                owner = lax.rem(my + 1 + s, N_DEV)
                rdma = chunk_copy(rs_src.at[l * 3 + s], rs.at[s],
                                  rs_send.at[l * 3 + s],
                                  rs_recv.at[l * 3 + s], owner)
                rdma.start()
                pending_sends.append(rdma)
            total = layer_partial(win, wout, xbuf[par, N_PEER])
            for s in range(N_PEER):
                wait_chunk_recv(rs.at[s], rs_recv.at[l * 3 + s])
            total = total + rs[0] + rs[1] + rs[2]

            if l < N_LAYERS - 1:
                nxt = 1 - par
                xbuf[nxt, N_PEER] = total
                for k in range(1, N_DEV):
                    peer = lax.rem(my + k, N_DEV)
                    s = N_PEER - k
                    rdma = chunk_copy(xbuf.at[nxt, N_PEER], xbuf.at[nxt, s],
                                      ag_send.at[(l + 1) * 3 + s],
                                      ag_recv.at[(l + 1) * 3 + s], peer)
                    rdma.start()
                    pending_sends.append(rdma)
            else:
                obuf[0] = total
                for k in range(1, N_DEV):
                    peer = lax.rem(my + k, N_DEV)
                    rdma = chunk_copy(obuf.at[0], obuf.at[N_DEV - k],
                                      out_send.at[k - 1],
                                      out_recv.at[k - 1], peer)
                    rdma.start()
                    pending_sends.append(rdma)
                out_ref[pl.ds(my * m_per, m_per), :] = total
                for r in range(1, N_DEV):
                    wait_chunk_recv(obuf.at[r], out_recv.at[N_PEER - r])
                    origin = lax.rem(my + r, N_DEV)
                    out_ref[pl.ds(origin * m_per, m_per), :] = obuf[r]

        for rdma in pending_sends:
            rdma.wait_send()

    return pl.pallas_call(
        body,
        out_shape=jax.ShapeDtypeStruct((m, d), jnp.float32),
        in_specs=[pl.BlockSpec(memory_space=pltpu.VMEM)] * 7,
        out_specs=pl.BlockSpec(memory_space=pltpu.VMEM),
        scratch_shapes=[
            pltpu.VMEM((2, N_DEV, m_per, d), jnp.float32),
            pltpu.VMEM((N_PEER, m_per, d), jnp.float32),
            pltpu.VMEM((N_LAYERS * N_PEER, m_per, d), jnp.float32),
            pltpu.VMEM((N_DEV, m_per, d), jnp.float32),
            pltpu.SemaphoreType.DMA((N_LAYERS * N_PEER,)),
            pltpu.SemaphoreType.DMA((N_LAYERS * N_PEER,)),
            pltpu.SemaphoreType.DMA((N_LAYERS * N_PEER,)),
            pltpu.SemaphoreType.DMA((N_LAYERS * N_PEER,)),
            pltpu.SemaphoreType.DMA((N_PEER,)),
            pltpu.SemaphoreType.DMA((N_PEER,)),
        ],
        compiler_params=pltpu.CompilerParams(collective_id=0),
    )(x, Win0, Wout0, Win1, Wout1, Win2, Wout2)
